# baseline (speedup 1.0000x reference)
import functools

import jax
import jax.numpy as jnp
import numpy as np
from jax import lax
from jax.experimental import pallas as pl
from jax.experimental.pallas import tpu as pltpu
from jax.experimental.pallas import tpu_sc as plsc

N_NODES = 100000
DIM = 16
N_TICKS = 20
BATCH = 16384

_info = plsc.get_sparse_core_info()
NC, NS, L = _info.num_cores, _info.num_subcores, _info.num_lanes
NW = NC * NS
BW = BATCH // NW
CHUNK = 128
NCHUNK = BW // CHUNK
GROUPS = CHUNK // L

STEP = np.float32(1.0 / (N_TICKS - 1))
ROW = DIM * N_TICKS


def _body(src_h, dst_h, t_h, z_h, bias_h, out_h,
          src_v, dst_v, t_v, bias_v, srows, drows, out_v, sem):
    wid = lax.axis_index("s") * NC + lax.axis_index("c")
    pltpu.sync_copy(src_h.at[wid], src_v)
    pltpu.sync_copy(dst_h.at[wid], dst_v)
    pltpu.sync_copy(t_h.at[wid], t_v)
    pltpu.sync_copy(bias_h, bias_v)
    bias_vec = bias_v[...]
    iota = lax.iota(jnp.int32, L)

    for c in range(NCHUNK):
        cp_s = pltpu.async_copy(z_h.at[src_v.at[c]], srows, sem)
        cp_d = pltpu.async_copy(z_h.at[dst_v.at[c]], drows, sem)
        cp_s.wait()
        cp_d.wait()

        def group(g, carry, c=c):
            base = c * CHUNK + g * L
            tv = t_v[pl.ds(base, L)]
            q = tv / STEP
            ti = jnp.minimum(q.astype(jnp.int32), N_TICKS - 2)
            dt = lax.rem(tv, STEP) / STEP
            omdt = 1.0 - dt
            row = iota + g * L
            acc = jnp.zeros((L,), jnp.float32)
            for d in range(DIM):
                dvec = jnp.full((L,), d, jnp.int32)
                colc = ti
                coln = colc + 1
                s_cur = plsc.load_gather(srows, [row, dvec, colc])
                s_nxt = plsc.load_gather(srows, [row, dvec, coln])
                d_cur = plsc.load_gather(drows, [row, dvec, colc])
                d_nxt = plsc.load_gather(drows, [row, dvec, coln])
                zs = omdt * s_cur + dt * s_nxt
                zd = omdt * d_cur + dt * d_nxt
                df = zs - zd
                acc = acc + df * df
            out_v[pl.ds(base, L)] = bias_vec - acc
            return carry

        lax.fori_loop(0, GROUPS, group, 0)

    pltpu.sync_copy(out_v, out_h.at[wid])


@functools.partial(
    pl.kernel,
    mesh=plsc.VectorSubcoreMesh(core_axis_name="c", subcore_axis_name="s"),
    out_type=jax.ShapeDtypeStruct((NW, BW), jnp.float32),
    compiler_params=pltpu.CompilerParams(
        use_tc_tiling_on_sc=False, needs_layout_passes=False),
    scratch_types=[
        pltpu.VMEM((NCHUNK, CHUNK), jnp.int32),
        pltpu.VMEM((NCHUNK, CHUNK), jnp.int32),
        pltpu.VMEM((BW,), jnp.float32),
        pltpu.VMEM((L,), jnp.float32),
        pltpu.VMEM((CHUNK, DIM, N_TICKS), jnp.float32),
        pltpu.VMEM((CHUNK, DIM, N_TICKS), jnp.float32),
        pltpu.VMEM((BW,), jnp.float32),
        pltpu.SemaphoreType.DMA,
    ],
)
def _decode_kernel(src_h, dst_h, t_h, z_h, bias_h, out_h, *scratch):
    _body(src_h, dst_h, t_h, z_h, bias_h, out_h, *scratch)


def kernel(src, dst, t, z, bias):
    src3 = src.astype(jnp.int32).reshape(NW, NCHUNK, CHUNK)
    dst3 = dst.astype(jnp.int32).reshape(NW, NCHUNK, CHUNK)
    t2 = t.reshape(NW, BW)
    bias_vec = jnp.full((L,), bias, dtype=jnp.float32)
    out = _decode_kernel(src3, dst3, t2, z, bias_vec)
    return out.reshape(BATCH)


# speedup vs baseline: 7.3255x; 7.3255x over previous
import functools

import jax
import jax.numpy as jnp
import numpy as np
from jax import lax
from jax.experimental import pallas as pl
from jax.experimental.pallas import tpu as pltpu
from jax.experimental.pallas import tpu_sc as plsc

N_NODES = 100000
DIM = 16
N_TICKS = 20
BATCH = 16384

_info = plsc.get_sparse_core_info()
NC, NS, L = _info.num_cores, _info.num_subcores, _info.num_lanes
NW = NC * NS
BW = BATCH // NW
CHUNK = 128
NCHUNK = BW // CHUNK
GROUPS = CHUNK // L
NJ = 4 * DIM

STEP = np.float32(1.0 / (N_TICKS - 1))
KSTRIDE = DIM * N_NODES


def _body(src_h, dst_h, t_h, z_h, bias_h, out_h,
          src_v, dst_v, t_v, bias_v, idx_v, data_v, out_v, sem):
    wid = lax.axis_index("s") * NC + lax.axis_index("c")
    pltpu.sync_copy(src_h.at[wid], src_v)
    pltpu.sync_copy(dst_h.at[wid], dst_v)
    pltpu.sync_copy(t_h.at[wid], t_v)
    pltpu.sync_copy(bias_h, bias_v)
    bias_vec = bias_v[...]

    for c in range(NCHUNK):
        def build(g, carry, c=c):
            base = c * CHUNK + g * L
            tv = t_v[pl.ds(base, L)]
            ti = jnp.minimum((tv / STEP).astype(jnp.int32), N_TICKS - 2)
            kbase = ti * KSTRIDE
            sbase = kbase + src_v[pl.ds(base, L)]
            dbase = kbase + dst_v[pl.ds(base, L)]
            for side in range(2):
                nodebase = sbase if side == 0 else dbase
                for o in range(2):
                    for d in range(DIM):
                        j = (side * 2 + o) * DIM + d
                        off = o * KSTRIDE + d * N_NODES
                        idx_v[j, pl.ds(g * L, L)] = nodebase + off
            return carry

        lax.fori_loop(0, GROUPS, build, 0)

        copies = [
            pltpu.async_copy(z_h.at[idx_v.at[j]], data_v.at[j], sem)
            for j in range(NJ)
        ]
        for cp in copies:
            cp.wait()

        def compute(g, carry, c=c):
            base = c * CHUNK + g * L
            tv = t_v[pl.ds(base, L)]
            dt = lax.rem(tv, STEP) / STEP
            omdt = 1.0 - dt
            acc = jnp.zeros((L,), jnp.float32)
            for d in range(DIM):
                s_cur = data_v[d, pl.ds(g * L, L)]
                s_nxt = data_v[DIM + d, pl.ds(g * L, L)]
                d_cur = data_v[2 * DIM + d, pl.ds(g * L, L)]
                d_nxt = data_v[3 * DIM + d, pl.ds(g * L, L)]
                zs = omdt * s_cur + dt * s_nxt
                zd = omdt * d_cur + dt * d_nxt
                df = zs - zd
                acc = acc + df * df
            out_v[pl.ds(base, L)] = bias_vec - acc
            return carry

        lax.fori_loop(0, GROUPS, compute, 0)

    pltpu.sync_copy(out_v, out_h.at[wid])


@functools.partial(
    pl.kernel,
    mesh=plsc.VectorSubcoreMesh(core_axis_name="c", subcore_axis_name="s"),
    out_type=jax.ShapeDtypeStruct((NW, BW), jnp.float32),
    compiler_params=pltpu.CompilerParams(
        use_tc_tiling_on_sc=False, needs_layout_passes=False),
    scratch_types=[
        pltpu.VMEM((BW,), jnp.int32),
        pltpu.VMEM((BW,), jnp.int32),
        pltpu.VMEM((BW,), jnp.float32),
        pltpu.VMEM((L,), jnp.float32),
        pltpu.VMEM((NJ, CHUNK), jnp.int32),
        pltpu.VMEM((NJ, CHUNK), jnp.float32),
        pltpu.VMEM((BW,), jnp.float32),
        pltpu.SemaphoreType.DMA,
    ],
)
def _decode_kernel(src_h, dst_h, t_h, z_h, bias_h, out_h, *scratch):
    _body(src_h, dst_h, t_h, z_h, bias_h, out_h, *scratch)


def kernel(src, dst, t, z, bias):
    src2 = src.astype(jnp.int32).reshape(NW, BW)
    dst2 = dst.astype(jnp.int32).reshape(NW, BW)
    t2 = t.reshape(NW, BW)
    z1 = z.transpose(2, 1, 0).reshape(-1)
    bias_vec = jnp.full((L,), bias, dtype=jnp.float32)
    out = _decode_kernel(src2, dst2, t2, z1, bias_vec)
    return out.reshape(BATCH)


# speedup vs baseline: 7.4171x; 1.0125x over previous
import functools

import jax
import jax.numpy as jnp
import numpy as np
from jax import lax
from jax.experimental import pallas as pl
from jax.experimental.pallas import tpu as pltpu
from jax.experimental.pallas import tpu_sc as plsc

N_NODES = 100000
DIM = 16
N_TICKS = 20
BATCH = 16384

_info = plsc.get_sparse_core_info()
NC, NS, L = _info.num_cores, _info.num_subcores, _info.num_lanes
NW = NC * NS
BW = BATCH // NW
CHUNK = 128
NCHUNK = BW // CHUNK
GROUPS = BW // L
NJ = 4 * DIM

STEP = np.float32(1.0 / (N_TICKS - 1))
KSTRIDE = DIM * N_NODES


def _body(src_h, dst_h, t_h, z_h, bias_h, out_h,
          src_v, dst_v, t_v, bias_v, idx_v, data_v, out_v, sem0, sem1):
    wid = lax.axis_index("s") * NC + lax.axis_index("c")
    pltpu.sync_copy(src_h.at[wid], src_v)
    pltpu.sync_copy(dst_h.at[wid], dst_v)
    pltpu.sync_copy(t_h.at[wid], t_v)
    pltpu.sync_copy(bias_h, bias_v)
    bias_vec = bias_v[...]
    sems = (sem0, sem1)

    def build(g, carry):
        base = g * L
        tv = t_v[pl.ds(base, L)]
        ti = jnp.minimum((tv / STEP).astype(jnp.int32), N_TICKS - 2)
        kbase = ti * KSTRIDE
        sbase = kbase + src_v[pl.ds(base, L)]
        dbase = kbase + dst_v[pl.ds(base, L)]
        for side in range(2):
            nodebase = sbase if side == 0 else dbase
            for o in range(2):
                for d in range(DIM):
                    j = (side * 2 + o) * DIM + d
                    off = o * KSTRIDE + d * N_NODES
                    idx_v[j, pl.ds(base, L)] = nodebase + off
        return carry

    lax.fori_loop(0, GROUPS, build, 0)

    def fire(c):
        buf = c % 2
        return [
            pltpu.async_copy(
                z_h.at[idx_v.at[j, pl.ds(c * CHUNK, CHUNK)]],
                data_v.at[buf, j],
                sems[buf],
            )
            for j in range(NJ)
        ]

    inflight = {0: fire(0), 1: fire(1)}

    for c in range(NCHUNK):
        buf = c % 2
        for cp in inflight.pop(c % 2):
            cp.wait()

        def compute(g, carry, c=c, buf=buf):
            base = c * CHUNK + g * L
            tv = t_v[pl.ds(base, L)]
            dt = lax.rem(tv, STEP) / STEP
            omdt = 1.0 - dt
            acc = jnp.zeros((L,), jnp.float32)
            gofs = g * L
            for d in range(DIM):
                s_cur = data_v[buf, d, pl.ds(gofs, L)]
                s_nxt = data_v[buf, DIM + d, pl.ds(gofs, L)]
                d_cur = data_v[buf, 2 * DIM + d, pl.ds(gofs, L)]
                d_nxt = data_v[buf, 3 * DIM + d, pl.ds(gofs, L)]
                df = omdt * (s_cur - d_cur) + dt * (s_nxt - d_nxt)
                acc = acc + df * df
            out_v[pl.ds(base, L)] = bias_vec - acc
            return carry

        lax.fori_loop(0, GROUPS // NCHUNK, compute, 0)
        if c + 2 < NCHUNK:
            inflight[buf] = fire(c + 2)

    pltpu.sync_copy(out_v, out_h.at[wid])


@functools.partial(
    pl.kernel,
    mesh=plsc.VectorSubcoreMesh(core_axis_name="c", subcore_axis_name="s"),
    out_type=jax.ShapeDtypeStruct((NW, BW), jnp.float32),
    compiler_params=pltpu.CompilerParams(
        use_tc_tiling_on_sc=False, needs_layout_passes=False),
    scratch_types=[
        pltpu.VMEM((BW,), jnp.int32),
        pltpu.VMEM((BW,), jnp.int32),
        pltpu.VMEM((BW,), jnp.float32),
        pltpu.VMEM((L,), jnp.float32),
        pltpu.VMEM((NJ, BW), jnp.int32),
        pltpu.VMEM((2, NJ, CHUNK), jnp.float32),
        pltpu.VMEM((BW,), jnp.float32),
        pltpu.SemaphoreType.DMA,
        pltpu.SemaphoreType.DMA,
    ],
)
def _decode_kernel(src_h, dst_h, t_h, z_h, bias_h, out_h, *scratch):
    _body(src_h, dst_h, t_h, z_h, bias_h, out_h, *scratch)


def kernel(src, dst, t, z, bias):
    src2 = src.astype(jnp.int32).reshape(NW, BW)
    dst2 = dst.astype(jnp.int32).reshape(NW, BW)
    t2 = t.reshape(NW, BW)
    z1 = z.transpose(2, 1, 0).reshape(-1)
    bias_vec = jnp.full((L,), bias, dtype=jnp.float32)
    out = _decode_kernel(src2, dst2, t2, z1, bias_vec)
    return out.reshape(BATCH)
